# Initial kernel scaffold; baseline (speedup 1.0000x reference)
import jax
import jax.numpy as jnp
from jax.experimental import pallas as pl


def kernel(x, position_ids, embeddings):
    raise NotImplementedError("write your pallas kernel here")


# speedup vs baseline: 1.2399x; 1.2399x over previous
import functools

import jax
import jax.numpy as jnp
from jax import lax
from jax.experimental import pallas as pl
from jax.experimental.pallas import tpu as pltpu
from jax.experimental.pallas import tpu_sc as plsc

BATCH = 4
SEQ = 8192
D = 768
LANES = 16

N_ROWS = BATCH * SEQ
NC, NS = 2, 16
NW = NC * NS
ROWS_PER_W = N_ROWS // NW
CHUNK = 64
N_CHUNKS = ROWS_PER_W // CHUNK
D_VECS = D // LANES


def _body(x_hbm, idx_hbm, table_hbm, out_hbm, idx_v, rows_v, x_v, gsem, xsem):
    wid = lax.axis_index("s") * NC + lax.axis_index("c")
    base = wid * ROWS_PER_W

    pltpu.sync_copy(idx_hbm.at[pl.ds(base, ROWS_PER_W)], idx_v)

    def chunk_step(c, _):
        row0 = base + c * CHUNK
        gcp = pltpu.async_copy(
            table_hbm.at[idx_v.at[pl.ds(c * CHUNK, CHUNK)]], rows_v, gsem)
        xcp = pltpu.async_copy(x_hbm.at[pl.ds(row0, CHUNK)], x_v, xsem)
        gcp.wait()
        xcp.wait()

        def add_row(i, _):
            for j in range(D_VECS):
                sl = pl.ds(j * LANES, LANES)
                plsc.addupdate(rows_v.at[i, sl], x_v[i, sl])
            return 0

        lax.fori_loop(0, CHUNK, add_row, 0)
        pltpu.sync_copy(rows_v, out_hbm.at[pl.ds(row0, CHUNK)])
        return 0

    lax.fori_loop(0, N_CHUNKS, chunk_step, 0)


@jax.jit
def _run(x2d, idx, table):
    mesh = plsc.VectorSubcoreMesh(core_axis_name="c", subcore_axis_name="s")
    return pl.kernel(
        _body,
        out_type=jax.ShapeDtypeStruct((N_ROWS, D), jnp.float32),
        mesh=mesh,
        scratch_types=[
            pltpu.VMEM((ROWS_PER_W,), jnp.int32),
            pltpu.VMEM((CHUNK, D), jnp.float32),
            pltpu.VMEM((CHUNK, D), jnp.float32),
            pltpu.SemaphoreType.DMA,
            pltpu.SemaphoreType.DMA,
        ],
    )(x2d, idx, table)


def kernel(x, position_ids, embeddings):
    x2d = x.reshape(N_ROWS, D)
    idx = position_ids.astype(jnp.int32).reshape(N_ROWS)
    out = _run(x2d, idx, embeddings)
    return out.reshape(BATCH, SEQ, D)


# speedup vs baseline: 1.6397x; 1.3225x over previous
import jax
import jax.numpy as jnp
from jax import lax
from jax.experimental import pallas as pl
from jax.experimental.pallas import tpu as pltpu
from jax.experimental.pallas import tpu_sc as plsc

BATCH = 4
SEQ = 8192
D = 768
LANES = 16

N_ROWS = BATCH * SEQ
NC, NS = 2, 16
NW = NC * NS
ROWS_PER_W = N_ROWS // NW
CHUNK = 32
N_CHUNKS = ROWS_PER_W // CHUNK
HALF = N_CHUNKS // 2
D_VECS = D // LANES


def _body(x_hbm, idx_hbm, table_hbm, out_hbm,
          idx_v, rows0, rows1, x0, x1,
          g0, g1, xs0, xs1, o0, o1):
    wid = lax.axis_index("s") * NC + lax.axis_index("c")
    base = wid * ROWS_PER_W

    pltpu.sync_copy(idx_hbm.at[pl.ds(base, ROWS_PER_W)], idx_v)

    def start(c, rows_v, x_v, gsem, xsem):
        pltpu.async_copy(
            table_hbm.at[idx_v.at[pl.ds(c * CHUNK, CHUNK)]], rows_v, gsem)
        pltpu.async_copy(x_hbm.at[pl.ds(base + c * CHUNK, CHUNK)], x_v, xsem)

    def wait(c, rows_v, x_v, gsem, xsem):
        pltpu.make_async_copy(
            table_hbm.at[idx_v.at[pl.ds(c * CHUNK, CHUNK)]], rows_v,
            gsem).wait()
        pltpu.make_async_copy(
            x_hbm.at[pl.ds(base + c * CHUNK, CHUNK)], x_v, xsem).wait()

    def add_chunk(rows_v, x_v):
        def add_row(i, _):
            for j in range(D_VECS):
                sl = pl.ds(j * LANES, LANES)
                plsc.addupdate(rows_v.at[i, sl], x_v[i, sl])
            return 0

        lax.fori_loop(0, CHUNK, add_row, 0)

    def out_start(c, rows_v, osem):
        pltpu.async_copy(rows_v, out_hbm.at[pl.ds(base + c * CHUNK, CHUNK)],
                         osem)

    def out_wait(c, rows_v, osem):
        pltpu.make_async_copy(
            rows_v, out_hbm.at[pl.ds(base + c * CHUNK, CHUNK)], osem).wait()

    start(0, rows0, x0, g0, xs0)
    start(1, rows1, x1, g1, xs1)

    def pair_step(i, _):
        c0 = 2 * i
        c1 = c0 + 1
        wait(c0, rows0, x0, g0, xs0)
        add_chunk(rows0, x0)
        out_start(c0, rows0, o0)
        wait(c1, rows1, x1, g1, xs1)
        add_chunk(rows1, x1)
        out_start(c1, rows1, o1)
        out_wait(c0, rows0, o0)
        start(c0 + 2, rows0, x0, g0, xs0)
        out_wait(c1, rows1, o1)
        start(c1 + 2, rows1, x1, g1, xs1)
        return 0

    lax.fori_loop(0, HALF - 1, pair_step, 0)

    c0 = N_CHUNKS - 2
    c1 = N_CHUNKS - 1
    wait(c0, rows0, x0, g0, xs0)
    add_chunk(rows0, x0)
    out_start(c0, rows0, o0)
    wait(c1, rows1, x1, g1, xs1)
    add_chunk(rows1, x1)
    out_start(c1, rows1, o1)
    out_wait(c0, rows0, o0)
    out_wait(c1, rows1, o1)


@jax.jit
def _run(x2d, idx, table):
    mesh = plsc.VectorSubcoreMesh(core_axis_name="c", subcore_axis_name="s")
    return pl.kernel(
        _body,
        out_type=jax.ShapeDtypeStruct((N_ROWS, D), jnp.float32),
        mesh=mesh,
        scratch_types=[
            pltpu.VMEM((ROWS_PER_W,), jnp.int32),
            pltpu.VMEM((CHUNK, D), jnp.float32),
            pltpu.VMEM((CHUNK, D), jnp.float32),
            pltpu.VMEM((CHUNK, D), jnp.float32),
            pltpu.VMEM((CHUNK, D), jnp.float32),
            pltpu.SemaphoreType.DMA,
            pltpu.SemaphoreType.DMA,
            pltpu.SemaphoreType.DMA,
            pltpu.SemaphoreType.DMA,
            pltpu.SemaphoreType.DMA,
            pltpu.SemaphoreType.DMA,
        ],
    )(x2d, idx, table)


def kernel(x, position_ids, embeddings):
    x2d = x.reshape(N_ROWS, D)
    idx = position_ids.astype(jnp.int32).reshape(N_ROWS)
    out = _run(x2d, idx, embeddings)
    return out.reshape(BATCH, SEQ, D)
